# Initial kernel scaffold; baseline (speedup 1.0000x reference)
import jax
import jax.numpy as jnp
from jax.experimental import pallas as pl


def kernel(x, edge_index_0, edge_index_1, edge_index_2, edge_attr_0, edge_attr_1, edge_attr_2, W1, b1, g1, be1, Wl2, bl2, Wr2, g2, be2, Wl3, bl3, Wr3, g3, be3, Wm1, bm1, Wm2, bm2):
    raise NotImplementedError("write your pallas kernel here")


# speedup vs baseline: 2.8092x; 2.8092x over previous
import functools

import jax
import jax.numpy as jnp
from jax import lax
from jax.experimental import pallas as pl
from jax.experimental.pallas import tpu as pltpu
from jax.experimental.pallas import tpu_sc as plsc

N = 10000
E = 320000
D = 128
DE = 16
H = 128
TW = 144
NC = 2
NS = 16
NW = NC * NS
CH = 128
EPAD = 323584
PW = EPAD // NW
NCHUNK = PW // CH
TN = N + 8
NACC = 10240
RPT = NACC // NS
SLAB = 128
NSLAB = RPT // SLAB

_mesh = plsc.VectorSubcoreMesh(core_axis_name="c", subcore_axis_name="s")


def _zero_rows(ref, nrows, ncols):
    zv = jnp.zeros((16,), jnp.float32)

    def row(i, carry):
        def col(j, carry2):
            ref[i, pl.ds(j * 16, 16)] = zv
            return carry2
        return lax.fori_loop(0, ncols // 16, col, carry)

    lax.fori_loop(0, nrows, row, 0)


def _sc_body(table, src, dst, ea, outA, outB, rows, eab, sidx, didx, accA,
             accB, sem, with_ea):
    c = lax.axis_index("c")
    s = lax.axis_index("s")
    wid = s * NC + c
    base = s * RPT

    _zero_rows(rows, CH, TW)
    if with_ea:
        _zero_rows(eab, CH, DE)
    for k in range(NSLAB):
        r0 = base + k * SLAB
        pltpu.sync_copy(rows.at[pl.ds(0, SLAB)], accA.at[pl.ds(r0, SLAB)])
        if with_ea:
            pltpu.sync_copy(eab.at[pl.ds(0, SLAB)], accB.at[pl.ds(r0, SLAB)])
    plsc.subcore_barrier()

    def chunk(k, carry):
        off = pl.multiple_of(wid * PW + k * CH, CH)
        pltpu.sync_copy(src.at[pl.ds(off, CH)], sidx)
        pltpu.sync_copy(dst.at[pl.ds(off, CH)], didx)
        pltpu.async_copy(table.at[sidx], rows, sem).wait()
        pltpu.sync_copy(rows, accA.at[didx], add=True)
        if with_ea:
            pltpu.sync_copy(ea.at[pl.ds(off, CH)], eab)
            pltpu.sync_copy(eab, accB.at[didx], add=True)
        return carry

    lax.fori_loop(0, NCHUNK, chunk, 0)
    plsc.subcore_barrier()

    for k in range(NSLAB):
        r0 = base + k * SLAB
        pltpu.sync_copy(accA.at[pl.ds(r0, SLAB)], outA.at[c, pl.ds(r0, SLAB)])
        if with_ea:
            pltpu.sync_copy(accB.at[pl.ds(r0, SLAB)],
                            outB.at[c, pl.ds(r0, SLAB)])


@functools.partial(
    pl.kernel,
    out_type=(jax.ShapeDtypeStruct((NC, NACC, TW), jnp.float32),
              jax.ShapeDtypeStruct((NC, NACC, DE), jnp.float32)),
    mesh=_mesh,
    scratch_types=[
        pltpu.VMEM((CH, TW), jnp.float32),
        pltpu.VMEM((CH, DE), jnp.float32),
        pltpu.VMEM((CH,), jnp.int32),
        pltpu.VMEM((CH,), jnp.int32),
        pltpu.VMEM_SHARED((NACC, TW), jnp.float32),
        pltpu.VMEM_SHARED((NACC, DE), jnp.float32),
        pltpu.SemaphoreType.DMA,
    ],
    compiler_params=pltpu.CompilerParams(use_tc_tiling_on_sc=False),
)
def _sc_pass_ea(table, src, dst, ea, outA, outB, rows, eab, sidx, didx, accA,
                accB, sem):
    _sc_body(table, src, dst, ea, outA, outB, rows, eab, sidx, didx, accA,
             accB, sem, True)


@functools.partial(
    pl.kernel,
    out_type=jax.ShapeDtypeStruct((NC, NACC, TW), jnp.float32),
    mesh=_mesh,
    scratch_types=[
        pltpu.VMEM((CH, TW), jnp.float32),
        pltpu.VMEM((CH,), jnp.int32),
        pltpu.VMEM((CH,), jnp.int32),
        pltpu.VMEM_SHARED((NACC, TW), jnp.float32),
        pltpu.SemaphoreType.DMA,
    ],
    compiler_params=pltpu.CompilerParams(use_tc_tiling_on_sc=False),
)
def _sc_pass(table, src, dst, outA, rows, sidx, didx, accA, sem):
    _sc_body(table, src, dst, None, outA, None, rows, None, sidx, didx, accA,
             None, sem, False)


def _bn_relu(h, g, b):
    m = jnp.mean(h, axis=0, keepdims=True)
    d = h - m
    v = jnp.mean(d * d, axis=0, keepdims=True)
    return jnp.maximum(d * lax.rsqrt(v + 1e-5) * g + b, 0.0)


def _with_ones(h):
    return jnp.concatenate(
        [h, jnp.ones((N, 1), jnp.float32), jnp.zeros((N, TW - D - 1),
                                                     jnp.float32)], axis=1)


def _tc1_body(Ap, Bp, W1, b1, g1, be1, out):
    A = Ap[0, :N] + Ap[1, :N]
    B = Bp[0, :N] + Bp[1, :N]
    cnt = A[:, D:D + 1]
    msum = (jnp.dot(A[:, :D], W1[:D], preferred_element_type=jnp.float32)
            + jnp.dot(B, W1[D:], preferred_element_type=jnp.float32)
            + cnt * b1[...])
    h = jnp.maximum(msum / jnp.maximum(cnt, 1.0), 0.0)
    out[...] = _with_ones(_bn_relu(h, g1[...], be1[...]))


def _tc23_body(Ap, hprev, Wl, bl, Wr, g, be, out):
    A = Ap[0, :N] + Ap[1, :N]
    cnt = A[:, D:D + 1]
    agg = A[:, :D] / jnp.maximum(cnt, 1.0)
    h = (jnp.dot(agg, Wl[...], preferred_element_type=jnp.float32) + bl[...]
         + jnp.dot(hprev[:, :D], Wr[...], preferred_element_type=jnp.float32))
    out[...] = _with_ones(_bn_relu(h, g[...], be[...]))


def _tc3_body(Ap, hprev, x, Wl, bl, Wr, g, be, Wm1, bm1, Wm2, bm2, out):
    A = Ap[0, :N] + Ap[1, :N]
    cnt = A[:, D:D + 1]
    agg = A[:, :D] / jnp.maximum(cnt, 1.0)
    h = (jnp.dot(agg, Wl[...], preferred_element_type=jnp.float32) + bl[...]
         + jnp.dot(hprev[:, :D], Wr[...], preferred_element_type=jnp.float32))
    h3 = _bn_relu(h, g[...], be[...])
    z1 = jnp.maximum(
        jnp.dot(h3, Wm1[:H], preferred_element_type=jnp.float32)
        + jnp.dot(x[...], Wm1[H:], preferred_element_type=jnp.float32)
        + bm1[...], 0.0)
    z = jnp.dot(z1, Wm2[...], preferred_element_type=jnp.float32) + bm2[...]
    out[...] = z[:, 0]


_tc_params = pltpu.CompilerParams(vmem_limit_bytes=110 * 1024 * 1024)

_tc1 = pl.pallas_call(
    _tc1_body,
    out_shape=jax.ShapeDtypeStruct((N, TW), jnp.float32),
    compiler_params=_tc_params,
)

_tc23 = pl.pallas_call(
    _tc23_body,
    out_shape=jax.ShapeDtypeStruct((N, TW), jnp.float32),
    compiler_params=_tc_params,
)

_tc3 = pl.pallas_call(
    _tc3_body,
    out_shape=jax.ShapeDtypeStruct((N,), jnp.float32),
    compiler_params=_tc_params,
)


def _pad_edges(ei):
    src = jnp.concatenate([ei[0], jnp.full((EPAD - E,), N, jnp.int32)])
    dst = jnp.concatenate([ei[1], jnp.zeros((EPAD - E,), jnp.int32)])
    return src, dst


def _pad_table(t):
    return jnp.concatenate([t, jnp.zeros((TN - N, TW), jnp.float32)], axis=0)


def kernel(x, edge_index_0, edge_index_1, edge_index_2, edge_attr_0,
           edge_attr_1, edge_attr_2, W1, b1, g1, be1, Wl2, bl2, Wr2, g2, be2,
           Wl3, bl3, Wr3, g3, be3, Wm1, bm1, Wm2, bm2):
    src0, dst0 = _pad_edges(edge_index_0)
    src1, dst1 = _pad_edges(edge_index_1)
    src2, dst2 = _pad_edges(edge_index_2)
    ea0 = jnp.concatenate(
        [edge_attr_0, jnp.zeros((EPAD - E, DE), jnp.float32)], axis=0)

    xt = _pad_table(_with_ones(x))
    A0p, B0p = _sc_pass_ea(xt, src0, dst0, ea0)
    h1t = _tc1(A0p, B0p, W1, b1, g1, be1)
    A1p = _sc_pass(_pad_table(h1t), src1, dst1)
    h2t = _tc23(A1p, h1t, Wl2, bl2, Wr2, g2, be2)
    A2p = _sc_pass(_pad_table(h2t), src2, dst2)
    return _tc3(A2p, h2t, x, Wl3, bl3, Wr3, g3, be3, Wm1, bm1, Wm2, bm2)


# speedup vs baseline: 3.1527x; 1.1223x over previous
import functools

import jax
import jax.numpy as jnp
from jax import lax
from jax.experimental import pallas as pl
from jax.experimental.pallas import tpu as pltpu
from jax.experimental.pallas import tpu_sc as plsc

N = 10000
E = 320000
D = 128
DE = 16
H = 128
HD = D // 2
EW = 32
CNT = DE
NC = 2
NS = 16
CH = 128
NCHUNK = 160
EPAD = NS * NCHUNK * CH
PW = NCHUNK * CH
ECH = NCHUNK // 2
TN = N + 8
NACC = 10240
RPT = NACC // NS
SLAB = 128
NSLAB = RPT // SLAB
NBUF = 4
LEAD = 2

_mesh = plsc.VectorSubcoreMesh(core_axis_name="c", subcore_axis_name="s")


def _zero_buf(ref, nrows, ncols):
    zv = jnp.zeros((16,), jnp.float32)

    def row(i, carry):
        def col(j, carry2):
            ref[0, i, pl.ds(j * 16, 16)] = zv
            return carry2
        return lax.fori_loop(0, ncols // 16, col, carry)

    lax.fori_loop(0, nrows, row, 0)


@functools.partial(
    pl.kernel,
    out_type=(jax.ShapeDtypeStruct((NC, NACC, HD), jnp.float32),
              jax.ShapeDtypeStruct((NC, NACC, EW), jnp.float32)),
    mesh=_mesh,
    scratch_types=[
        pltpu.VMEM((NBUF, CH, HD), jnp.float32),
        pltpu.VMEM((NBUF, CH, EW), jnp.float32),
        pltpu.VMEM((NBUF, CH), jnp.int32),
        pltpu.VMEM((NBUF, CH), jnp.int32),
        pltpu.VMEM_SHARED((NACC, HD), jnp.float32),
        pltpu.VMEM_SHARED((NACC, EW), jnp.float32),
        pltpu.SemaphoreType.DMA((NBUF,)),
        pltpu.SemaphoreType.DMA((NBUF,)),
        pltpu.SemaphoreType.DMA((NBUF,)),
        pltpu.SemaphoreType.DMA((NBUF,)),
    ],
    compiler_params=pltpu.CompilerParams(use_tc_tiling_on_sc=False),
)
def _sc_pass(table, srcs, dsts, ea, outA, outB, rows, eab, sidx, didx, accA,
             accB, gsem, easem, ssem, dsem):
    c = lax.axis_index("c")
    s = lax.axis_index("s")
    base = s * RPT

    _zero_buf(rows, SLAB, HD)
    _zero_buf(eab, SLAB, EW)
    for k in range(NSLAB):
        r0 = base + k * SLAB
        pltpu.sync_copy(rows.at[0], accA.at[pl.ds(r0, SLAB)])
        pltpu.sync_copy(eab.at[0], accB.at[pl.ds(r0, SLAB)])
    plsc.subcore_barrier()

    def i_start(k):
        b = k % NBUF
        pltpu.async_copy(srcs.at[c, s * NCHUNK + k], sidx.at[b], ssem.at[b])
        pltpu.async_copy(dsts.at[s * NCHUNK + k], didx.at[b], dsem.at[b])

    def i_wait(k):
        b = k % NBUF
        pltpu.make_async_copy(srcs.at[c, 0], sidx.at[b], ssem.at[b]).wait()
        pltpu.make_async_copy(dsts.at[0], didx.at[b], dsem.at[b]).wait()

    def g_start(k):
        b = k % NBUF
        pltpu.async_copy(table.at[sidx.at[b]], rows.at[b], gsem.at[b])

    def g_wait(k):
        b = k % NBUF
        pltpu.make_async_copy(table.at[sidx.at[0]], rows.at[b],
                              gsem.at[b]).wait()

    def e_start(k):
        b = k % NBUF
        off = pl.multiple_of(s * PW + k * CH, CH)
        pltpu.async_copy(ea.at[pl.ds(off, CH)], eab.at[b], easem.at[b])

    def e_wait(k):
        b = k % NBUF
        pltpu.make_async_copy(ea.at[pl.ds(0, CH)], eab.at[b],
                              easem.at[b]).wait()

    def s_sync(k):
        b = k % NBUF
        pltpu.sync_copy(rows.at[b], accA.at[didx.at[b]], add=True)

    def e_sync(k):
        b = k % NBUF
        pltpu.sync_copy(eab.at[b], accB.at[didx.at[b]], add=True)

    def step(k, ea_cond, idx_w, g_next, idx_n, ea_w, ea_n, ea_s):
        if idx_w:
            i_wait(k + LEAD)
        g_wait(k)
        if ea_w or ea_n:
            @pl.when(ea_cond)
            def _ea_in():
                if ea_w:
                    e_wait(k)
                if ea_n:
                    e_start(k + LEAD)
        if g_next:
            g_start(k + LEAD)
        if idx_n:
            i_start(k + LEAD + 1)
        s_sync(k)
        if ea_s:
            @pl.when(ea_cond)
            def _ea_out():
                e_sync(k)

    for k in range(LEAD + 1):
        i_start(k)
    for k in range(LEAD):
        i_wait(k)
        g_start(k)

    @pl.when(c == 0)
    def _prime_ea_a():
        for k in range(LEAD):
            e_start(k)

    on0 = c == 0

    def group_a(i, carry):
        for b in range(NBUF):
            k = i * NBUF + b
            step(k, on0, True, True, True, True, True, True)
        return carry

    lax.fori_loop(0, (ECH - NBUF) // NBUF, group_a, 0)
    for k in range(ECH - NBUF, ECH - LEAD):
        step(k, on0, True, True, True, True, True, True)
    for k in range(ECH - LEAD, ECH):
        step(k, on0, True, True, True, True, False, True)

    on1 = c == 1

    @pl.when(on1)
    def _prime_ea_b():
        for k in range(ECH, ECH + LEAD):
            e_start(k)

    def group_b(i, carry):
        for b in range(NBUF):
            k = ECH + i * NBUF + b
            step(k, on1, True, True, True, True, True, True)
        return carry

    lax.fori_loop(0, (NCHUNK - ECH - NBUF) // NBUF, group_b, 0)
    step(NCHUNK - 4, on1, True, True, True, True, True, True)
    step(NCHUNK - 3, on1, True, True, False, True, True, True)
    step(NCHUNK - 2, on1, False, False, False, True, False, True)
    step(NCHUNK - 1, on1, False, False, False, True, False, True)

    plsc.subcore_barrier()

    for k in range(NSLAB):
        r0 = base + k * SLAB
        pltpu.sync_copy(accA.at[pl.ds(r0, SLAB)], outA.at[c, pl.ds(r0, SLAB)])
        pltpu.sync_copy(accB.at[pl.ds(r0, SLAB)], outB.at[c, pl.ds(r0, SLAB)])


def _bn_relu(h, g, b):
    m = jnp.mean(h, axis=0, keepdims=True)
    d = h - m
    v = jnp.mean(d * d, axis=0, keepdims=True)
    return jnp.maximum(d * lax.rsqrt(v + 1e-5) * g + b, 0.0)


def _tc1_body(Ap, Bp, W1, b1, g1, be1, out):
    A = jnp.concatenate([Ap[0, :N], Ap[1, :N]], axis=1)
    B = Bp[0, :N] + Bp[1, :N]
    cnt = B[:, CNT:CNT + 1]
    msum = (jnp.dot(A, W1[:D], preferred_element_type=jnp.float32)
            + jnp.dot(B[:, :DE], W1[D:], preferred_element_type=jnp.float32)
            + cnt * b1[...])
    h = jnp.maximum(msum / jnp.maximum(cnt, 1.0), 0.0)
    out[...] = _bn_relu(h, g1[...], be1[...])


def _tc23_body(Ap, Bp, hprev, Wl, bl, Wr, g, be, out):
    A = jnp.concatenate([Ap[0, :N], Ap[1, :N]], axis=1)
    cnt = Bp[0, :N, CNT:CNT + 1] + Bp[1, :N, CNT:CNT + 1]
    agg = A / jnp.maximum(cnt, 1.0)
    h = (jnp.dot(agg, Wl[...], preferred_element_type=jnp.float32) + bl[...]
         + jnp.dot(hprev[...], Wr[...], preferred_element_type=jnp.float32))
    out[...] = _bn_relu(h, g[...], be[...])


def _tc3_body(Ap, Bp, hprev, x, Wl, bl, Wr, g, be, Wm1, bm1, Wm2, bm2, out):
    A = jnp.concatenate([Ap[0, :N], Ap[1, :N]], axis=1)
    cnt = Bp[0, :N, CNT:CNT + 1] + Bp[1, :N, CNT:CNT + 1]
    agg = A / jnp.maximum(cnt, 1.0)
    h = (jnp.dot(agg, Wl[...], preferred_element_type=jnp.float32) + bl[...]
         + jnp.dot(hprev[...], Wr[...], preferred_element_type=jnp.float32))
    h3 = _bn_relu(h, g[...], be[...])
    z1 = jnp.maximum(
        jnp.dot(h3, Wm1[:H], preferred_element_type=jnp.float32)
        + jnp.dot(x[...], Wm1[H:], preferred_element_type=jnp.float32)
        + bm1[...], 0.0)
    z = jnp.dot(z1, Wm2[...], preferred_element_type=jnp.float32) + bm2[...]
    out[...] = z[:, 0]


_tc_params = pltpu.CompilerParams(vmem_limit_bytes=110 * 1024 * 1024)

_tc1 = pl.pallas_call(
    _tc1_body,
    out_shape=jax.ShapeDtypeStruct((N, D), jnp.float32),
    compiler_params=_tc_params,
)

_tc23 = pl.pallas_call(
    _tc23_body,
    out_shape=jax.ShapeDtypeStruct((N, D), jnp.float32),
    compiler_params=_tc_params,
)

_tc3 = pl.pallas_call(
    _tc3_body,
    out_shape=jax.ShapeDtypeStruct((N,), jnp.float32),
    compiler_params=_tc_params,
)


def _pad_edges(ei):
    src = jnp.concatenate([ei[0], jnp.full((EPAD - E,), N, jnp.int32)])
    dst = jnp.concatenate([ei[1], jnp.zeros((EPAD - E,), jnp.int32)])
    srcs = jnp.stack([src, src + TN]).reshape(NC, EPAD // CH, CH)
    return srcs, dst.reshape(EPAD // CH, CH)


def _edge_stream(attr):
    return jnp.concatenate(
        [attr, jnp.ones((E, 1), jnp.float32),
         jnp.zeros((E, EW - DE - 1), jnp.float32)], axis=1)


def _pad_rows(t, nrows):
    return jnp.concatenate(
        [t, jnp.zeros((nrows - t.shape[0], t.shape[1]), jnp.float32)], axis=0)


def _split_table(t):
    return jnp.concatenate(
        [_pad_rows(t[:, :HD], TN), _pad_rows(t[:, HD:], TN)], axis=0)


def kernel(x, edge_index_0, edge_index_1, edge_index_2, edge_attr_0,
           edge_attr_1, edge_attr_2, W1, b1, g1, be1, Wl2, bl2, Wr2, g2, be2,
           Wl3, bl3, Wr3, g3, be3, Wm1, bm1, Wm2, bm2):
    src0, dst0 = _pad_edges(edge_index_0)
    src1, dst1 = _pad_edges(edge_index_1)
    src2, dst2 = _pad_edges(edge_index_2)
    ea0 = _pad_rows(_edge_stream(edge_attr_0), EPAD)
    ones_ea = _pad_rows(_edge_stream(jnp.zeros((E, DE), jnp.float32)), EPAD)

    A0p, B0p = _sc_pass(_split_table(x), src0, dst0, ea0)
    h1 = _tc1(A0p, B0p, W1, b1, g1, be1)
    A1p, B1p = _sc_pass(_split_table(h1), src1, dst1, ones_ea)
    h2 = _tc23(A1p, B1p, h1, Wl2, bl2, Wr2, g2, be2)
    A2p, B2p = _sc_pass(_split_table(h2), src2, dst2, ones_ea)
    return _tc3(A2p, B2p, h2, x, Wl3, bl3, Wr3, g3, be3, Wm1, bm1, Wm2, bm2)


# speedup vs baseline: 3.2379x; 1.0270x over previous
import functools

import jax
import jax.numpy as jnp
from jax import lax
from jax.experimental import pallas as pl
from jax.experimental.pallas import tpu as pltpu
from jax.experimental.pallas import tpu_sc as plsc

N = 10000
E = 320000
D = 128
DE = 16
H = 128
HD = D // 2
EW = 32
CNT = DE
NC = 2
NS = 16
CH = 128
NCHUNK = 160
EPAD = NS * NCHUNK * CH
PW = NCHUNK * CH
ECH = NCHUNK // 2
TN = N + 8
NACC = 10240
RPT = NACC // NS
SLAB = 128
NSLAB = RPT // SLAB
NBUF = 4
LEAD = 2

_mesh = plsc.VectorSubcoreMesh(core_axis_name="c", subcore_axis_name="s")


def _zero_buf(ref, nrows, ncols):
    zv = jnp.zeros((16,), jnp.float32)

    def row(i, carry):
        def col(j, carry2):
            ref[0, i, pl.ds(j * 16, 16)] = zv
            return carry2
        return lax.fori_loop(0, ncols // 16, col, carry)

    lax.fori_loop(0, nrows, row, 0)


@functools.partial(
    pl.kernel,
    out_type=(jax.ShapeDtypeStruct((NC, NACC, HD), jnp.float32),
              jax.ShapeDtypeStruct((NC, NACC, EW), jnp.float32)),
    mesh=_mesh,
    scratch_types=[
        pltpu.VMEM((NBUF, CH, HD), jnp.float32),
        pltpu.VMEM((NBUF, CH, EW), jnp.float32),
        pltpu.VMEM((NBUF, CH), jnp.int32),
        pltpu.VMEM((NBUF, CH), jnp.int32),
        pltpu.VMEM_SHARED((NACC, HD), jnp.float32),
        pltpu.VMEM_SHARED((NACC, EW), jnp.float32),
        pltpu.SemaphoreType.DMA((NBUF,)),
        pltpu.SemaphoreType.DMA((NBUF,)),
        pltpu.SemaphoreType.DMA((NBUF,)),
        pltpu.SemaphoreType.DMA((NBUF,)),
    ],
    compiler_params=pltpu.CompilerParams(use_tc_tiling_on_sc=False),
)
def _sc_pass(table, srcs, dsts, ea, outA, outB, rows, eab, sidx, didx, accA,
             accB, gsem, easem, ssem, dsem):
    c = lax.axis_index("c")
    s = lax.axis_index("s")
    base = s * RPT

    _zero_buf(rows, SLAB, HD)
    _zero_buf(eab, SLAB, EW)
    for k in range(NSLAB):
        r0 = base + k * SLAB
        pltpu.sync_copy(rows.at[0], accA.at[pl.ds(r0, SLAB)])
        pltpu.sync_copy(eab.at[0], accB.at[pl.ds(r0, SLAB)])
    plsc.subcore_barrier()

    def i_start(k):
        b = k % NBUF
        pltpu.async_copy(srcs.at[c, s * NCHUNK + k], sidx.at[b], ssem.at[b])
        pltpu.async_copy(dsts.at[s * NCHUNK + k], didx.at[b], dsem.at[b])

    def i_wait(k):
        b = k % NBUF
        pltpu.make_async_copy(srcs.at[c, 0], sidx.at[b], ssem.at[b]).wait()
        pltpu.make_async_copy(dsts.at[0], didx.at[b], dsem.at[b]).wait()

    def g_start(k):
        b = k % NBUF
        pltpu.async_copy(table.at[sidx.at[b]], rows.at[b], gsem.at[b])

    def g_wait(k):
        b = k % NBUF
        pltpu.make_async_copy(table.at[sidx.at[0]], rows.at[b],
                              gsem.at[b]).wait()

    def e_start(k):
        b = k % NBUF
        off = pl.multiple_of(s * PW + k * CH, CH)
        pltpu.async_copy(ea.at[pl.ds(off, CH)], eab.at[b], easem.at[b])

    def e_wait(k):
        b = k % NBUF
        pltpu.make_async_copy(ea.at[pl.ds(0, CH)], eab.at[b],
                              easem.at[b]).wait()

    def s_sync(k):
        b = k % NBUF
        pltpu.sync_copy(rows.at[b], accA.at[didx.at[b]], add=True)

    def e_sync(k):
        b = k % NBUF
        pltpu.sync_copy(eab.at[b], accB.at[didx.at[b]], add=True)

    def step(k, ea_cond, idx_w, g_next, idx_n, ea_w, ea_n, ea_s):
        if idx_w:
            i_wait(k + LEAD)
        g_wait(k)
        if ea_w or ea_n:
            @pl.when(ea_cond)
            def _ea_in():
                if ea_w:
                    e_wait(k)
                if ea_n:
                    e_start(k + LEAD)
        if g_next:
            g_start(k + LEAD)
        if idx_n:
            i_start(k + LEAD + 1)
        s_sync(k)
        if ea_s:
            @pl.when(ea_cond)
            def _ea_out():
                e_sync(k)

    for k in range(LEAD + 1):
        i_start(k)
    for k in range(LEAD):
        i_wait(k)
        g_start(k)

    @pl.when(c == 0)
    def _prime_ea_a():
        for k in range(LEAD):
            e_start(k)

    on0 = c == 0

    def group_a(i, carry):
        for b in range(NBUF):
            k = i * NBUF + b
            step(k, on0, True, True, True, True, True, True)
        return carry

    lax.fori_loop(0, (ECH - NBUF) // NBUF, group_a, 0)
    for k in range(ECH - NBUF, ECH - LEAD):
        step(k, on0, True, True, True, True, True, True)
    for k in range(ECH - LEAD, ECH):
        step(k, on0, True, True, True, True, False, True)

    on1 = c == 1

    @pl.when(on1)
    def _prime_ea_b():
        for k in range(ECH, ECH + LEAD):
            e_start(k)

    def group_b(i, carry):
        for b in range(NBUF):
            k = ECH + i * NBUF + b
            step(k, on1, True, True, True, True, True, True)
        return carry

    lax.fori_loop(0, (NCHUNK - ECH - NBUF) // NBUF, group_b, 0)
    step(NCHUNK - 4, on1, True, True, True, True, True, True)
    step(NCHUNK - 3, on1, True, True, False, True, True, True)
    step(NCHUNK - 2, on1, False, False, False, True, False, True)
    step(NCHUNK - 1, on1, False, False, False, True, False, True)

    plsc.subcore_barrier()

    for k in range(NSLAB):
        r0 = base + k * SLAB
        pltpu.sync_copy(accA.at[pl.ds(r0, SLAB)], outA.at[c, pl.ds(r0, SLAB)])
        pltpu.sync_copy(accB.at[pl.ds(r0, SLAB)], outB.at[c, pl.ds(r0, SLAB)])


def _bn_relu(h, g, b):
    m = jnp.mean(h, axis=0, keepdims=True)
    d = h - m
    v = jnp.mean(d * d, axis=0, keepdims=True)
    return jnp.maximum(d * lax.rsqrt(v + 1e-5) * g + b, 0.0)


def _split_out(h, out):
    pad = jnp.zeros((TN - N, HD), jnp.float32)
    out[...] = jnp.stack(
        [jnp.concatenate([h[:, :HD], pad], axis=0),
         jnp.concatenate([h[:, HD:], pad], axis=0)])


def _tc1_body(Ap, Bp, W1, b1, g1, be1, out):
    A = jnp.concatenate([Ap[0, :N], Ap[1, :N]], axis=1)
    B = Bp[0, :N] + Bp[1, :N]
    cnt = B[:, CNT:CNT + 1]
    msum = (jnp.dot(A, W1[:D], preferred_element_type=jnp.float32)
            + jnp.dot(B[:, :DE], W1[D:], preferred_element_type=jnp.float32)
            + cnt * b1[...])
    h = jnp.maximum(msum / jnp.maximum(cnt, 1.0), 0.0)
    _split_out(_bn_relu(h, g1[...], be1[...]), out)


def _tc23_body(Ap, Bp, hsplit, Wl, bl, Wr, g, be, out):
    A = jnp.concatenate([Ap[0, :N], Ap[1, :N]], axis=1)
    hprev = jnp.concatenate([hsplit[0, :N], hsplit[1, :N]], axis=1)
    cnt = Bp[0, :N, CNT:CNT + 1] + Bp[1, :N, CNT:CNT + 1]
    agg = A / jnp.maximum(cnt, 1.0)
    h = (jnp.dot(agg, Wl[...], preferred_element_type=jnp.float32) + bl[...]
         + jnp.dot(hprev, Wr[...], preferred_element_type=jnp.float32))
    _split_out(_bn_relu(h, g[...], be[...]), out)


def _tc3_body(Ap, Bp, hsplit, x, Wl, bl, Wr, g, be, Wm1, bm1, Wm2, bm2, out):
    A = jnp.concatenate([Ap[0, :N], Ap[1, :N]], axis=1)
    hprev = jnp.concatenate([hsplit[0, :N], hsplit[1, :N]], axis=1)
    cnt = Bp[0, :N, CNT:CNT + 1] + Bp[1, :N, CNT:CNT + 1]
    agg = A / jnp.maximum(cnt, 1.0)
    h = (jnp.dot(agg, Wl[...], preferred_element_type=jnp.float32) + bl[...]
         + jnp.dot(hprev, Wr[...], preferred_element_type=jnp.float32))
    h3 = _bn_relu(h, g[...], be[...])
    z1 = jnp.maximum(
        jnp.dot(h3, Wm1[:H], preferred_element_type=jnp.float32)
        + jnp.dot(x[...], Wm1[H:], preferred_element_type=jnp.float32)
        + bm1[...], 0.0)
    z = jnp.dot(z1, Wm2[...], preferred_element_type=jnp.float32) + bm2[...]
    out[...] = z[:, 0]


_tc_params = pltpu.CompilerParams(vmem_limit_bytes=110 * 1024 * 1024)

_split_shape = jax.ShapeDtypeStruct((NC, TN, HD), jnp.float32)

_tc1 = pl.pallas_call(
    _tc1_body,
    out_shape=_split_shape,
    compiler_params=_tc_params,
)

_tc23 = pl.pallas_call(
    _tc23_body,
    out_shape=_split_shape,
    compiler_params=_tc_params,
)

_tc3 = pl.pallas_call(
    _tc3_body,
    out_shape=jax.ShapeDtypeStruct((N,), jnp.float32),
    compiler_params=_tc_params,
)


def _pad_edges(ei):
    src = jnp.concatenate([ei[0], jnp.full((EPAD - E,), N, jnp.int32)])
    dst = jnp.concatenate([ei[1], jnp.zeros((EPAD - E,), jnp.int32)])
    srcs = jnp.stack([src, src + TN]).reshape(NC, EPAD // CH, CH)
    return srcs, dst.reshape(EPAD // CH, CH)


def _edge_stream(attr):
    return jnp.concatenate(
        [attr, jnp.ones((E, 1), jnp.float32),
         jnp.zeros((E, EW - DE - 1), jnp.float32)], axis=1)


def _pad_rows(t, nrows):
    return jnp.concatenate(
        [t, jnp.zeros((nrows - t.shape[0], t.shape[1]), jnp.float32)], axis=0)


def _split_table(t):
    return jnp.stack([_pad_rows(t[:, :HD], TN), _pad_rows(t[:, HD:], TN)])


def kernel(x, edge_index_0, edge_index_1, edge_index_2, edge_attr_0,
           edge_attr_1, edge_attr_2, W1, b1, g1, be1, Wl2, bl2, Wr2, g2, be2,
           Wl3, bl3, Wr3, g3, be3, Wm1, bm1, Wm2, bm2):
    src0, dst0 = _pad_edges(edge_index_0)
    src1, dst1 = _pad_edges(edge_index_1)
    src2, dst2 = _pad_edges(edge_index_2)
    ea0 = _pad_rows(_edge_stream(edge_attr_0), EPAD)
    ones_ea = _pad_rows(_edge_stream(jnp.zeros((E, DE), jnp.float32)), EPAD)

    def flat(t):
        return t.reshape(NC * TN, HD)

    A0p, B0p = _sc_pass(flat(_split_table(x)), src0, dst0, ea0)
    h1s = _tc1(A0p, B0p, W1, b1, g1, be1)
    A1p, B1p = _sc_pass(flat(h1s), src1, dst1, ones_ea)
    h2s = _tc23(A1p, B1p, h1s, Wl2, bl2, Wr2, g2, be2)
    A2p, B2p = _sc_pass(flat(h2s), src2, dst2, ones_ea)
    return _tc3(A2p, B2p, h2s, x, Wl3, bl3, Wr3, g3, be3, Wm1, bm1, Wm2, bm2)
